# baseline (speedup 1.0000x reference)
import jax
import jax.numpy as jnp
from jax.experimental import pallas as pl

N = 50000
E = 1600000
B = 512
F = 32
K = 16
CUTOFF = 6.0
N_ITER = 2

_ROWS = 2000


def _silu_mm(x, W):
    n = x.shape[0]
    grid = n // _ROWS

    def body(x_ref, w_ref, o_ref):
        h = jnp.dot(x_ref[...], w_ref[...], preferred_element_type=jnp.float32)
        o_ref[...] = h * jax.nn.sigmoid(h)

    return pl.pallas_call(
        body,
        grid=(grid,),
        in_specs=[
            pl.BlockSpec((_ROWS, F), lambda i: (i, 0)),
            pl.BlockSpec((F, F), lambda i: (0, 0)),
        ],
        out_specs=pl.BlockSpec((_ROWS, F), lambda i: (i, 0)),
        out_shape=jax.ShapeDtypeStruct((n, F), jnp.float32),
    )(x, W)


def _smooth_cutoff(r, cutoff):
    x = r / cutoff
    x3 = x ** 3
    x4 = x3 * x
    x5 = x4 * x
    f = 1.0 - 6.0 * x5 + 15.0 * x4 - 10.0 * x3
    return jnp.where(r < cutoff, f, 0.0)


def kernel(atomic_numbers, positions, dst_idx, src_idx, batch_segments,
           graph_mask, total_charges, total_spins,
           charge_embed_W, spin_embed_W, atom_embed_W, mol_proj_W,
           rbf_W, msg_W, upd_W, out_W):
    ci = jnp.clip((total_charges - (-5)).astype(jnp.int32), 0, 10)
    si = jnp.clip((total_spins - 1).astype(jnp.int32), 0, 6)
    charge_feat = jnp.take(charge_embed_W, ci, axis=0)
    spin_feat = jnp.take(spin_embed_W, si, axis=0)
    mol_features = jnp.concatenate([charge_feat, spin_feat], axis=-1)
    mol_per_atom = mol_features[batch_segments]
    x = jnp.take(atom_embed_W, atomic_numbers, axis=0) + mol_per_atom @ mol_proj_W

    disp = positions[src_idx] - positions[dst_idx]
    r = jnp.sqrt(jnp.sum(disp * disp, axis=-1) + 1e-12)
    centers = jnp.linspace(0.0, CUTOFF, K)
    rbf = jnp.exp(-4.0 * (r[:, None] - centers[None, :]) ** 2)
    rbf = rbf * _smooth_cutoff(r, CUTOFF)[:, None]

    for it in range(N_ITER):
        gate = rbf @ rbf_W[it]
        msg = gate * _silu_mm(x, msg_W[it])[src_idx]
        agg = jax.ops.segment_sum(msg, dst_idx, num_segments=N)
        x = x + _silu_mm(agg, upd_W[it])

    e_atom = (x @ out_W)[:, 0]
    energy = jax.ops.segment_sum(e_atom, batch_segments, num_segments=B)
    return energy * graph_mask.astype(energy.dtype)
